# baseline (speedup 1.0000x reference)
import functools

import jax
import jax.numpy as jnp
from jax import lax
from jax.experimental import pallas as pl
from jax.experimental.pallas import tpu as pltpu
from jax.experimental.pallas import tpu_sc as plsc

N_SEG = 10000
N_ROWS = 320000
D = 128

NC = 2
NS = 16
NW = NC * NS
ROWS_PER_W = N_ROWS // NW
CHUNK = 80
NCHUNK = ROWS_PER_W // CHUNK
NBUF = 3
STEADY = (NCHUNK // NBUF - 1) * NBUF
N_SEG_PAD = 10240
SEG_PER_TILE = N_SEG_PAD // NS


def _sc_partial_sums(x, batch2d, zeros):
    mesh = plsc.VectorSubcoreMesh(core_axis_name="c", subcore_axis_name="s")

    @functools.partial(
        pl.kernel,
        out_type=jax.ShapeDtypeStruct((NC, N_SEG_PAD, D), jnp.float32),
        mesh=mesh,
        scratch_types=[
            pltpu.VMEM((NCHUNK, CHUNK), jnp.int32),
            pltpu.VMEM((NBUF, CHUNK, D), jnp.float32),
            pltpu.VMEM_SHARED((N_SEG_PAD, D), jnp.float32),
        ] + [pltpu.SemaphoreType.DMA] * (2 * NBUF),
    )
    def body(x_hbm, b_hbm, z_hbm, out_hbm, ids_v, rows_v, acc, *sems):
        fsems, asems = sems[:NBUF], sems[NBUF:]
        c = lax.axis_index("c")
        s = lax.axis_index("s")
        wid = c * NS + s
        row_base = wid * ROWS_PER_W

        def fetch(chunk, b):
            pltpu.async_copy(
                x_hbm.at[pl.ds(row_base + chunk * CHUNK, CHUNK)],
                rows_v.at[b], fsems[b])

        def wait_fetch(b):
            pltpu.make_async_copy(
                x_hbm.at[pl.ds(0, CHUNK)], rows_v.at[b], fsems[b]).wait()

        def add_start(t, b):
            pltpu.async_copy(rows_v.at[b], acc.at[ids_v.at[t]], asems[b],
                             add=True)

        def wait_add(b):
            pltpu.make_async_copy(rows_v.at[b], acc.at[ids_v.at[0]],
                                  asems[b]).wait()

        for b in range(NBUF):
            fetch(b, b)

        pltpu.sync_copy(z_hbm, acc.at[pl.ds(s * SEG_PER_TILE, SEG_PER_TILE)])
        pltpu.sync_copy(b_hbm.at[wid], ids_v)
        plsc.subcore_barrier()

        wait_fetch(0)
        add_start(0, 0)
        for t in (1, 2):
            wait_fetch(t)
            add_start(t, t)
            wait_add(t - 1)
            fetch(t + NBUF - 1, t - 1)

        @pl.loop(NBUF, NCHUNK - 2, step=NBUF)
        def _(j):
            for b in range(NBUF):
                wait_fetch(b)
                add_start(j + b, b)
                bp = (b - 1) % NBUF
                wait_add(bp)
                fetch(j + b + NBUF - 1, bp)

        for t in range(NCHUNK - 2, NCHUNK):
            b = t % NBUF
            wait_fetch(b)
            add_start(t, b)
            wait_add((b - 1) % NBUF)
        wait_add((NCHUNK - 1) % NBUF)

        plsc.subcore_barrier()
        pltpu.sync_copy(acc.at[pl.ds(s * SEG_PER_TILE, SEG_PER_TILE)],
                        out_hbm.at[c, pl.ds(s * SEG_PER_TILE, SEG_PER_TILE)])

    return body(x, batch2d, zeros)


def _merge_body(p_ref, o_ref):
    o_ref[...] = p_ref[0] + p_ref[1]


def _merge(partials):
    blk = 1000
    return pl.pallas_call(
        _merge_body,
        grid=(N_SEG // blk,),
        in_specs=[pl.BlockSpec((NC, blk, D), lambda i: (0, i, 0))],
        out_specs=pl.BlockSpec((blk, D), lambda i: (i, 0)),
        out_shape=jax.ShapeDtypeStruct((N_SEG, D), jnp.float32),
    )(partials)


def kernel(x, batch):
    batch3d = batch.reshape(NW, NCHUNK, CHUNK)
    zeros = jnp.zeros((SEG_PER_TILE, D), jnp.float32)
    partials = _sc_partial_sums(x, batch3d, zeros)
    return _merge(partials)


# speedup vs baseline: 1.0879x; 1.0879x over previous
import functools

import jax
import jax.numpy as jnp
from jax import lax
from jax.experimental import pallas as pl
from jax.experimental.pallas import tpu as pltpu
from jax.experimental.pallas import tpu_sc as plsc

N_SEG = 10000
N_ROWS = 320000
D = 128

NC = 2
NS = 16
NW = NC * NS
ROWS_PER_W = N_ROWS // NW
CHUNK = 80
NCHUNK = ROWS_PER_W // CHUNK
NBUF = 3
STEADY = (NCHUNK // NBUF - 1) * NBUF
N_SEG_PAD = 10240
SEG_PER_TILE = N_SEG_PAD // NS


def _sc_partial_sums(x, batch2d, zeros):
    mesh = plsc.VectorSubcoreMesh(core_axis_name="c", subcore_axis_name="s")

    @functools.partial(
        pl.kernel,
        out_type=jax.ShapeDtypeStruct((NC, N_SEG_PAD, D), jnp.float32),
        mesh=mesh,
        scratch_types=[
            pltpu.VMEM((NCHUNK, CHUNK), jnp.int32),
            pltpu.VMEM((NBUF, CHUNK, D), jnp.float32),
            pltpu.VMEM_SHARED((N_SEG_PAD, D), jnp.float32),
        ] + [pltpu.SemaphoreType.DMA] * NBUF,
    )
    def body(x_hbm, b_hbm, z_hbm, out_hbm, ids_v, rows_v, acc, *sems):
        c = lax.axis_index("c")
        s = lax.axis_index("s")
        wid = c * NS + s
        row_base = wid * ROWS_PER_W

        for b in range(NBUF):
            pltpu.async_copy(
                x_hbm.at[pl.ds(row_base + b * CHUNK, CHUNK)],
                rows_v.at[b], sems[b])

        pltpu.sync_copy(z_hbm, acc.at[pl.ds(s * SEG_PER_TILE, SEG_PER_TILE)])
        pltpu.sync_copy(b_hbm.at[wid], ids_v)
        plsc.subcore_barrier()

        def wait_fetch(b):
            pltpu.make_async_copy(
                x_hbm.at[pl.ds(0, CHUNK)], rows_v.at[b], sems[b]).wait()

        @pl.loop(0, STEADY, step=NBUF)
        def _(j):
            for b in range(NBUF):
                wait_fetch(b)
                pltpu.sync_copy(rows_v.at[b], acc.at[ids_v.at[j + b]],
                                add=True)
                pltpu.async_copy(
                    x_hbm.at[pl.ds(row_base + (j + b + NBUF) * CHUNK, CHUNK)],
                    rows_v.at[b], sems[b])

        for t in range(STEADY, NCHUNK):
            b = t % NBUF
            wait_fetch(b)
            pltpu.sync_copy(rows_v.at[b], acc.at[ids_v.at[t]], add=True)
            if t + NBUF < NCHUNK:
                pltpu.async_copy(
                    x_hbm.at[pl.ds(row_base + (t + NBUF) * CHUNK, CHUNK)],
                    rows_v.at[b], sems[b])

        plsc.subcore_barrier()
        pltpu.sync_copy(acc.at[pl.ds(s * SEG_PER_TILE, SEG_PER_TILE)],
                        out_hbm.at[c, pl.ds(s * SEG_PER_TILE, SEG_PER_TILE)])

    return body(x, batch2d, zeros)


def _merge_body(p_ref, o_ref):
    o_ref[...] = p_ref[0] + p_ref[1]


def _merge(partials):
    blk = 1000
    return pl.pallas_call(
        _merge_body,
        grid=(N_SEG // blk,),
        in_specs=[pl.BlockSpec((NC, blk, D), lambda i: (0, i, 0))],
        out_specs=pl.BlockSpec((blk, D), lambda i: (i, 0)),
        out_shape=jax.ShapeDtypeStruct((N_SEG, D), jnp.float32),
    )(partials)


def kernel(x, batch):
    batch3d = batch.reshape(NW, NCHUNK, CHUNK)
    zeros = jnp.zeros((SEG_PER_TILE, D), jnp.float32)
    partials = _sc_partial_sums(x, batch3d, zeros)
    return _merge(partials)


# speedup vs baseline: 1.1099x; 1.0202x over previous
import functools

import jax
import jax.numpy as jnp
from jax import lax
from jax.experimental import pallas as pl
from jax.experimental.pallas import tpu as pltpu
from jax.experimental.pallas import tpu_sc as plsc

N_SEG = 10000
N_ROWS = 320000
D = 128

NC = 2
NS = 16
NW = NC * NS
ROWS_PER_W = N_ROWS // NW
CHUNK = 80
NCHUNK = ROWS_PER_W // CHUNK
NBUF = 3
STEADY = (NCHUNK // NBUF - 1) * NBUF
N_SEG_PAD = 10240
SEG_PER_TILE = N_SEG_PAD // NS


def _sc_partial_sums(x, batch2d, zeros):
    mesh = plsc.VectorSubcoreMesh(core_axis_name="c", subcore_axis_name="s")

    @functools.partial(
        pl.kernel,
        out_type=jax.ShapeDtypeStruct((NC, N_SEG_PAD, D), jnp.float32),
        mesh=mesh,
        scratch_types=[
            pltpu.VMEM((NCHUNK, CHUNK), jnp.int32),
            pltpu.VMEM((NBUF, CHUNK, D), jnp.float32),
            pltpu.VMEM_SHARED((N_SEG_PAD, D), jnp.float32),
        ] + [pltpu.SemaphoreType.DMA] * NBUF,
    )
    def body(x_hbm, b_hbm, z_hbm, out_hbm, ids_v, rows_v, acc, *sems):
        c = lax.axis_index("c")
        s = lax.axis_index("s")
        wid = c * NS + s
        row_base = wid * ROWS_PER_W

        for b in range(NBUF):
            pltpu.async_copy(
                x_hbm.at[pl.ds(row_base + b * CHUNK, CHUNK)],
                rows_v.at[b], sems[b])

        pltpu.sync_copy(z_hbm, acc.at[pl.ds(s * SEG_PER_TILE, SEG_PER_TILE)])
        pltpu.sync_copy(b_hbm.at[wid], ids_v)
        plsc.subcore_barrier()

        def wait_fetch(b):
            pltpu.make_async_copy(
                x_hbm.at[pl.ds(0, CHUNK)], rows_v.at[b], sems[b]).wait()

        @pl.loop(0, STEADY, step=NBUF)
        def _(j):
            for b in range(NBUF):
                wait_fetch(b)
                pltpu.sync_copy(rows_v.at[b], acc.at[ids_v.at[j + b]],
                                add=True)
                pltpu.async_copy(
                    x_hbm.at[pl.ds(row_base + (j + b + NBUF) * CHUNK, CHUNK)],
                    rows_v.at[b], sems[b])

        for t in range(STEADY, NCHUNK):
            b = t % NBUF
            wait_fetch(b)
            pltpu.sync_copy(rows_v.at[b], acc.at[ids_v.at[t]], add=True)
            if t + NBUF < NCHUNK:
                pltpu.async_copy(
                    x_hbm.at[pl.ds(row_base + (t + NBUF) * CHUNK, CHUNK)],
                    rows_v.at[b], sems[b])

        plsc.subcore_barrier()
        pltpu.sync_copy(acc.at[pl.ds(s * SEG_PER_TILE, SEG_PER_TILE)],
                        out_hbm.at[c, pl.ds(s * SEG_PER_TILE, SEG_PER_TILE)])

    return body(x, batch2d, zeros)


def _merge_body(p_ref, o_ref):
    o_ref[...] = p_ref[0] + p_ref[1]


def _merge(partials):
    blk = 2000
    return pl.pallas_call(
        _merge_body,
        grid=(N_SEG // blk,),
        in_specs=[pl.BlockSpec((NC, blk, D), lambda i: (0, i, 0))],
        out_specs=pl.BlockSpec((blk, D), lambda i: (i, 0)),
        out_shape=jax.ShapeDtypeStruct((N_SEG, D), jnp.float32),
    )(partials)


def kernel(x, batch):
    batch3d = batch.reshape(NW, NCHUNK, CHUNK)
    zeros = jnp.zeros((SEG_PER_TILE, D), jnp.float32)
    partials = _sc_partial_sums(x, batch3d, zeros)
    return _merge(partials)


# speedup vs baseline: 1.1207x; 1.0098x over previous
import functools

import jax
import jax.numpy as jnp
from jax import lax
from jax.experimental import pallas as pl
from jax.experimental.pallas import tpu as pltpu
from jax.experimental.pallas import tpu_sc as plsc

N_SEG = 10000
N_ROWS = 320000
D = 128

NC = 2
NS = 16
NW = NC * NS
ROWS_PER_W = N_ROWS // NW
CHUNK = 80
NCHUNK = ROWS_PER_W // CHUNK
NBUF = 3
STEADY = (NCHUNK // NBUF - 1) * NBUF
N_SEG_PAD = 10240
SEG_PER_TILE = N_SEG_PAD // NS


def _sc_partial_sums(x, batch2d, zeros):
    mesh = plsc.VectorSubcoreMesh(core_axis_name="c", subcore_axis_name="s")

    @functools.partial(
        pl.kernel,
        out_type=jax.ShapeDtypeStruct((NC, N_SEG_PAD, D), jnp.float32),
        mesh=mesh,
        scratch_types=[
            pltpu.VMEM((NCHUNK, CHUNK), jnp.int32),
            pltpu.VMEM((NBUF, CHUNK, D), jnp.float32),
            pltpu.VMEM_SHARED((N_SEG_PAD, D), jnp.float32),
        ] + [pltpu.SemaphoreType.DMA] * NBUF,
    )
    def body(x_hbm, b_hbm, z_hbm, out_hbm, ids_v, rows_v, acc, *sems):
        c = lax.axis_index("c")
        s = lax.axis_index("s")
        wid = c * NS + s
        row_base = wid * ROWS_PER_W

        for b in range(NBUF):
            pltpu.async_copy(
                x_hbm.at[pl.ds(row_base + b * CHUNK, CHUNK)],
                rows_v.at[b], sems[b])

        pltpu.sync_copy(z_hbm, acc.at[pl.ds(s * SEG_PER_TILE, SEG_PER_TILE)])
        pltpu.sync_copy(b_hbm.at[wid], ids_v)
        plsc.subcore_barrier()

        def wait_fetch(b):
            pltpu.make_async_copy(
                x_hbm.at[pl.ds(0, CHUNK)], rows_v.at[b], sems[b]).wait()

        @pl.loop(0, STEADY, step=NBUF)
        def _(j):
            for b in range(NBUF):
                wait_fetch(b)
                pltpu.sync_copy(rows_v.at[b], acc.at[ids_v.at[j + b]],
                                add=True)
                pltpu.async_copy(
                    x_hbm.at[pl.ds(row_base + (j + b + NBUF) * CHUNK, CHUNK)],
                    rows_v.at[b], sems[b])

        for t in range(STEADY, NCHUNK):
            b = t % NBUF
            wait_fetch(b)
            pltpu.sync_copy(rows_v.at[b], acc.at[ids_v.at[t]], add=True)
            if t + NBUF < NCHUNK:
                pltpu.async_copy(
                    x_hbm.at[pl.ds(row_base + (t + NBUF) * CHUNK, CHUNK)],
                    rows_v.at[b], sems[b])

        plsc.subcore_barrier()
        pltpu.sync_copy(acc.at[pl.ds(s * SEG_PER_TILE, SEG_PER_TILE)],
                        out_hbm.at[c, pl.ds(s * SEG_PER_TILE, SEG_PER_TILE)])

    return body(x, batch2d, zeros)


def _merge_body(p_ref, o_ref):
    o_ref[...] = p_ref[0] + p_ref[1]


def _merge(partials):
    blk = 5000
    return pl.pallas_call(
        _merge_body,
        grid=(N_SEG // blk,),
        in_specs=[pl.BlockSpec((NC, blk, D), lambda i: (0, i, 0))],
        out_specs=pl.BlockSpec((blk, D), lambda i: (i, 0)),
        out_shape=jax.ShapeDtypeStruct((N_SEG, D), jnp.float32),
    )(partials)


def kernel(x, batch):
    batch3d = batch.reshape(NW, NCHUNK, CHUNK)
    zeros = jnp.zeros((SEG_PER_TILE, D), jnp.float32)
    partials = _sc_partial_sums(x, batch3d, zeros)
    return _merge(partials)


# speedup vs baseline: 1.1788x; 1.0519x over previous
import functools

import jax
import jax.numpy as jnp
from jax import lax
from jax.experimental import pallas as pl
from jax.experimental.pallas import tpu as pltpu
from jax.experimental.pallas import tpu_sc as plsc

N_SEG = 10000
N_ROWS = 320000
D = 128

NC = 2
NS = 16
NW = NC * NS
ROWS_PER_W = N_ROWS // NW
CHUNK = 80
NCHUNK = ROWS_PER_W // CHUNK
NBUF = 3
STEADY = (NCHUNK // NBUF - 1) * NBUF
N_SEG_PAD = 10240
SEG_PER_TILE = N_SEG_PAD // NS


def _sc_partial_sums(x, batch2d):
    mesh = plsc.VectorSubcoreMesh(core_axis_name="c", subcore_axis_name="s")

    @functools.partial(
        pl.kernel,
        out_type=jax.ShapeDtypeStruct((NC, N_SEG_PAD, D), jnp.float32),
        mesh=mesh,
        scratch_types=[
            pltpu.VMEM((NCHUNK, CHUNK), jnp.int32),
            pltpu.VMEM((NBUF, CHUNK, D), jnp.float32),
            pltpu.VMEM_SHARED((N_SEG_PAD, D), jnp.float32),
        ] + [pltpu.SemaphoreType.DMA] * (NBUF + 1),
    )
    def body(x_hbm, b_hbm, out_hbm, ids_v, rows_v, acc, *sems):
        zsem = sems[NBUF]
        c = lax.axis_index("c")
        s = lax.axis_index("s")
        wid = c * NS + s
        row_base = wid * ROWS_PER_W

        for b in range(1, NBUF):
            pltpu.async_copy(
                x_hbm.at[pl.ds(row_base + b * CHUNK, CHUNK)],
                rows_v.at[b], sems[b])

        z16 = jnp.zeros((16,), jnp.float32)

        @pl.loop(0, CHUNK)
        def _(i):
            for g in range(D // 16):
                rows_v[0, i, pl.ds(g * 16, 16)] = z16

        for k in range(SEG_PER_TILE // CHUNK):
            pltpu.async_copy(
                rows_v.at[0],
                acc.at[pl.ds(s * SEG_PER_TILE + k * CHUNK, CHUNK)], zsem)
        pltpu.sync_copy(b_hbm.at[wid], ids_v)
        for k in range(SEG_PER_TILE // CHUNK):
            pltpu.make_async_copy(
                rows_v.at[0], acc.at[pl.ds(0, CHUNK)], zsem).wait()
        pltpu.async_copy(x_hbm.at[pl.ds(row_base, CHUNK)],
                         rows_v.at[0], sems[0])
        plsc.subcore_barrier()

        def wait_fetch(b):
            pltpu.make_async_copy(
                x_hbm.at[pl.ds(0, CHUNK)], rows_v.at[b], sems[b]).wait()

        @pl.loop(0, STEADY, step=NBUF)
        def _(j):
            for b in range(NBUF):
                wait_fetch(b)
                pltpu.sync_copy(rows_v.at[b], acc.at[ids_v.at[j + b]],
                                add=True)
                pltpu.async_copy(
                    x_hbm.at[pl.ds(row_base + (j + b + NBUF) * CHUNK, CHUNK)],
                    rows_v.at[b], sems[b])

        for t in range(STEADY, NCHUNK):
            b = t % NBUF
            wait_fetch(b)
            pltpu.sync_copy(rows_v.at[b], acc.at[ids_v.at[t]], add=True)
            if t + NBUF < NCHUNK:
                pltpu.async_copy(
                    x_hbm.at[pl.ds(row_base + (t + NBUF) * CHUNK, CHUNK)],
                    rows_v.at[b], sems[b])

        plsc.subcore_barrier()
        pltpu.sync_copy(acc.at[pl.ds(s * SEG_PER_TILE, SEG_PER_TILE)],
                        out_hbm.at[c, pl.ds(s * SEG_PER_TILE, SEG_PER_TILE)])

    return body(x, batch2d)


def _merge_body(p_ref, o_ref):
    o_ref[...] = p_ref[0] + p_ref[1]


def _merge(partials):
    blk = 5000
    return pl.pallas_call(
        _merge_body,
        grid=(N_SEG // blk,),
        in_specs=[pl.BlockSpec((NC, blk, D), lambda i: (0, i, 0))],
        out_specs=pl.BlockSpec((blk, D), lambda i: (i, 0)),
        out_shape=jax.ShapeDtypeStruct((N_SEG, D), jnp.float32),
    )(partials)


def kernel(x, batch):
    batch3d = batch.reshape(NW, NCHUNK, CHUNK)
    partials = _sc_partial_sums(x, batch3d)
    return _merge(partials)
